# baseline (speedup 1.0000x reference)
import functools

import jax
import jax.numpy as jnp
from jax import lax
from jax.experimental import pallas as pl
from jax.experimental.pallas import tpu as pltpu
from jax.experimental.pallas import tpu_sc as plsc

N = 100000
E = 3200000
NC = 2
NS = 16
NW = NC * NS
NPAD = 102400
TSL = NPAD // NS
G = 125
WB = 100
D = 5

A_GROUPS = E // NW // G
B_CH = 16
B_GPC = E // NS // B_CH // G
C_CH = 64
C_GPC = E // NS // C_CH // G

_mesh = plsc.VectorSubcoreMesh(core_axis_name="c", subcore_axis_name="s")
_sc_params = pltpu.CompilerParams(use_tc_tiling_on_sc=False)


def _zero_fill(ref, rows):
    def body(i, _):
        ref[i] = jnp.zeros((16,), jnp.float32)
        return 0
    lax.fori_loop(0, rows, body, 0)


def _zero_fill_1d(ref, n16):
    def body(i, _):
        ref[pl.ds(i * 16, 16)] = jnp.zeros((16,), jnp.float32)
        return 0
    lax.fori_loop(0, n16, body, 0)


def _edge_loop(src_hbm, dst_hbm, sidx, gather_fn, scatter_fn,
               srcb, dstb, gsem, ssem, isem, nch, gpc):
    pltpu.async_copy(src_hbm.at[sidx, 0], srcb.at[0], isem.at[0])
    pltpu.async_copy(dst_hbm.at[sidx, 0], dstb.at[0], isem.at[1])

    def chunk(k, _):
        b = lax.rem(k, 2)
        pltpu.make_async_copy(src_hbm.at[sidx, k], srcb.at[b],
                              isem.at[0]).wait()
        pltpu.make_async_copy(dst_hbm.at[sidx, k], dstb.at[b],
                              isem.at[1]).wait()
        @pl.when(k + 1 < nch)
        def _():
            pltpu.async_copy(src_hbm.at[sidx, k + 1], srcb.at[1 - b],
                             isem.at[0])
            pltpu.async_copy(dst_hbm.at[sidx, k + 1], dstb.at[1 - b],
                             isem.at[1])
        def quint(q, _):
            gs = [gather_fn(srcb.at[b, q * D + d], d) for d in range(D)]
            ss = []
            for d in range(D):
                gs[d].wait()
                ss.append(scatter_fn(dstb.at[b, q * D + d], d))
            for d in range(D):
                ss[d].wait()
            return 0
        lax.fori_loop(0, gpc // D, quint, 0)
        return 0
    lax.fori_loop(0, nch, chunk, 0)


def _deg_body(dst_hbm, out_hbm, acc, dst_v, ones_v, zb_v, osem):
    c = lax.axis_index("c")
    s = lax.axis_index("s")
    w = c * NS + s
    _zero_fill_1d(zb_v, TSL // 16)
    pltpu.sync_copy(zb_v, acc.at[pl.ds(s * TSL, TSL)])
    def ones_body(i, _):
        ones_v[pl.ds(i * 16, 16)] = jnp.ones((16,), jnp.float32)
        return 0
    lax.fori_loop(0, 8, ones_body, 0)
    plsc.subcore_barrier()
    pltpu.sync_copy(dst_hbm.at[w], dst_v)
    def grp(t, _):
        cs = [pltpu.async_copy(ones_v.at[pl.ds(0, G)],
                               acc.at[dst_v.at[t * 8 + d]],
                               osem.at[d], add=True)
              for d in range(8)]
        for d in range(8):
            cs[d].wait()
        return 0
    lax.fori_loop(0, A_GROUPS // 8, grp, 0)
    plsc.subcore_barrier()
    pltpu.sync_copy(acc.at[pl.ds(s * TSL, TSL)], zb_v)
    pltpu.sync_copy(zb_v, out_hbm.at[c, pl.ds(s * TSL, TSL)])


@functools.partial(
    pl.kernel,
    out_type=jax.ShapeDtypeStruct((NC, NPAD), jnp.float32),
    mesh=_mesh,
    compiler_params=_sc_params,
    scratch_types=[
        pltpu.VMEM_SHARED((NPAD,), jnp.float32),
        pltpu.VMEM((A_GROUPS, G), jnp.int32),
        pltpu.VMEM((128,), jnp.float32),
        pltpu.VMEM((TSL,), jnp.float32),
        pltpu.SemaphoreType.DMA((8,)),
    ],
)
def _sc_degree(dst_hbm, out_hbm, acc, dst_v, ones_v, zb_v, osem):
    _deg_body(dst_hbm, out_hbm, acc, dst_v, ones_v, zb_v, osem)


def _p1_body(y1c_hbm, src_hbm, dst_hbm, out_hbm, tbl, acc,
             srcb, dstb, rows_v, yb_v, zb_v, gsem, ssem, isem):
    c = lax.axis_index("c")
    s = lax.axis_index("s")
    pltpu.sync_copy(y1c_hbm.at[c, pl.ds(s * TSL, TSL)], yb_v)
    pltpu.sync_copy(yb_v, tbl.at[pl.ds(s * TSL, TSL)])
    _zero_fill_1d(zb_v, TSL // 16)
    pltpu.sync_copy(zb_v, acc.at[pl.ds(s * TSL, TSL)])
    plsc.subcore_barrier()
    def gather(idx, d):
        return pltpu.async_copy(tbl.at[idx], rows_v.at[d], gsem.at[d])
    def scatter(idx, d):
        return pltpu.async_copy(rows_v.at[d], acc.at[idx], ssem.at[d],
                                add=True)
    _edge_loop(src_hbm, dst_hbm, s, gather, scatter,
               srcb, dstb, gsem, ssem, isem, B_CH, B_GPC)
    plsc.subcore_barrier()
    pltpu.sync_copy(acc.at[pl.ds(s * TSL, TSL)], zb_v)
    pltpu.sync_copy(zb_v, out_hbm.at[c, pl.ds(s * TSL, TSL)])


@functools.partial(
    pl.kernel,
    out_type=jax.ShapeDtypeStruct((NC, NPAD), jnp.float32),
    mesh=_mesh,
    compiler_params=_sc_params,
    scratch_types=[
        pltpu.VMEM_SHARED((NPAD,), jnp.float32),
        pltpu.VMEM_SHARED((NPAD,), jnp.float32),
        pltpu.VMEM((2, B_GPC, G), jnp.int32),
        pltpu.VMEM((2, B_GPC, G), jnp.int32),
        pltpu.VMEM((D, G), jnp.float32),
        pltpu.VMEM((TSL,), jnp.float32),
        pltpu.VMEM((TSL,), jnp.float32),
        pltpu.SemaphoreType.DMA((D,)),
        pltpu.SemaphoreType.DMA((D,)),
        pltpu.SemaphoreType.DMA((2,)),
    ],
)
def _sc_prop1(y1c_hbm, src_hbm, dst_hbm, out_hbm, tbl, acc,
              srcb, dstb, rows_v, yb_v, zb_v, gsem, ssem, isem):
    _p1_body(y1c_hbm, src_hbm, dst_hbm, out_hbm, tbl, acc,
             srcb, dstb, rows_v, yb_v, zb_v, gsem, ssem, isem)


def _p2_body(y2v_hbm, src_hbm, dst_hbm, out_hbm, acc,
             srcb, dstb, rows_v, zb_v, gsem, ssem, isem):
    c = lax.axis_index("c")
    s = lax.axis_index("s")
    tbl = y2v_hbm.at[pl.ds(c, 2 * NPAD)]
    _zero_fill(zb_v, WB)
    def zinit(t, _):
        pltpu.sync_copy(zb_v, acc.at[pl.ds(s * TSL + t * WB, WB)])
        return 0
    lax.fori_loop(0, TSL // WB, zinit, 0)
    plsc.subcore_barrier()
    def gather(idx, d):
        return pltpu.async_copy(tbl.at[idx], rows_v.at[d], gsem.at[d])
    def scatter(idx, d):
        return pltpu.async_copy(rows_v.at[d], acc.at[idx], ssem.at[d],
                                add=True)
    _edge_loop(src_hbm, dst_hbm, s, gather, scatter,
               srcb, dstb, gsem, ssem, isem, C_CH, C_GPC)
    plsc.subcore_barrier()
    def wback(t, _):
        pltpu.sync_copy(acc.at[pl.ds(s * TSL + t * WB, WB)], zb_v)
        pltpu.sync_copy(zb_v, out_hbm.at[pl.ds(s * TSL + t * WB, WB), c])
        return 0
    lax.fori_loop(0, TSL // WB, wback, 0)


@functools.partial(
    pl.kernel,
    out_type=jax.ShapeDtypeStruct((NPAD, NC, 16), jnp.float32),
    mesh=_mesh,
    compiler_params=_sc_params,
    scratch_types=[
        pltpu.VMEM_SHARED((NPAD, 16), jnp.float32),
        pltpu.VMEM((2, C_GPC, G), jnp.int32),
        pltpu.VMEM((2, C_GPC, G), jnp.int32),
        pltpu.VMEM((D, G, 16), jnp.float32),
        pltpu.VMEM((WB, 16), jnp.float32),
        pltpu.SemaphoreType.DMA((D,)),
        pltpu.SemaphoreType.DMA((D,)),
        pltpu.SemaphoreType.DMA((2,)),
    ],
)
def _sc_prop16(y2v_hbm, src_hbm, dst_hbm, out_hbm, acc,
               srcb, dstb, rows_v, zb_v, gsem, ssem, isem):
    _p2_body(y2v_hbm, src_hbm, dst_hbm, out_hbm, acc,
             srcb, dstb, rows_v, zb_v, gsem, ssem, isem)


BLK = 6400
GRID = NPAD // BLK
BL8 = BLK // 8


def _tc_prep_body(deg2_ref, x_ref, dinv_ref, y1c_ref):
    d = deg2_ref[0, :] + deg2_ref[1, :] + 1.0
    dv = lax.rsqrt(d)
    dinv_ref[0, :] = dv
    y1c_ref[0, :] = x_ref[:, 0] * dv
    y1c_ref[1, :] = x_ref[:, 1] * dv


def _tc_prep(deg2, xp):
    return pl.pallas_call(
        _tc_prep_body,
        grid=(GRID,),
        in_specs=[
            pl.BlockSpec((NC, BLK), lambda i: (0, i)),
            pl.BlockSpec((BLK, 2), lambda i: (i, 0)),
        ],
        out_specs=[
            pl.BlockSpec((1, BLK), lambda i: (0, i)),
            pl.BlockSpec((NC, BLK), lambda i: (0, i)),
        ],
        out_shape=[
            jax.ShapeDtypeStruct((1, NPAD), jnp.float32),
            jax.ShapeDtypeStruct((NC, NPAD), jnp.float32),
        ],
    )(deg2, xp)


def _tc_mid_body(p1c_ref, y1c_ref, dinv_ref, W1_ref, b1_ref, y2_ref):
    dv = dinv_ref[0, :]
    prop0 = dv * (p1c_ref[0, :] + y1c_ref[0, :])
    prop1 = dv * (p1c_ref[1, :] + y1c_ref[1, :])
    h = (prop0[:, None] * W1_ref[0:1, :]
         + prop1[:, None] * W1_ref[1:2, :]
         + b1_ref[...])
    h = jnp.maximum(h, 0.0)
    y2_ref[...] = h * dv[:, None]


def _tc_mid(p1c, y1c, dinv, W1, b1):
    return pl.pallas_call(
        _tc_mid_body,
        grid=(GRID,),
        in_specs=[
            pl.BlockSpec((NC, BLK), lambda i: (0, i)),
            pl.BlockSpec((NC, BLK), lambda i: (0, i)),
            pl.BlockSpec((1, BLK), lambda i: (0, i)),
            pl.BlockSpec((2, 32), lambda i: (0, 0)),
            pl.BlockSpec((1, 32), lambda i: (0, 0)),
        ],
        out_specs=pl.BlockSpec((BLK, 32), lambda i: (i, 0)),
        out_shape=jax.ShapeDtypeStruct((NPAD, 32), jnp.float32),
    )(p1c, y1c, dinv, W1, b1.reshape(1, 32))


def _tc_final_body(p2_ref, y2_ref, dinv_ref, W2_ref, b2_ref,
                   Wf1_ref, bf1_ref, Wf2_ref, bf2_ref, out_ref):
    dv = dinv_ref[0, :][:, None]
    prop2 = dv * (p2_ref[...] + y2_ref[...])
    h2 = jnp.maximum(
        jnp.dot(prop2, W2_ref[...], preferred_element_type=jnp.float32)
        + b2_ref[...], 0.0)
    h3 = jnp.maximum(
        jnp.dot(h2, Wf1_ref[...], preferred_element_type=jnp.float32)
        + bf1_ref[...], 0.0)
    out_ref[...] = (
        jnp.dot(h3, Wf2_ref[...], preferred_element_type=jnp.float32)
        + bf2_ref[...])


def _tc_final(p2, y2, dinv, W2, b2, Wf1, bf1, Wf2, bf2):
    return pl.pallas_call(
        _tc_final_body,
        grid=(GRID,),
        in_specs=[
            pl.BlockSpec((BLK, 32), lambda i: (i, 0)),
            pl.BlockSpec((BLK, 32), lambda i: (i, 0)),
            pl.BlockSpec((1, BLK), lambda i: (0, i)),
            pl.BlockSpec((32, 32), lambda i: (0, 0)),
            pl.BlockSpec((1, 32), lambda i: (0, 0)),
            pl.BlockSpec((32, 32), lambda i: (0, 0)),
            pl.BlockSpec((1, 32), lambda i: (0, 0)),
            pl.BlockSpec((32, 1), lambda i: (0, 0)),
            pl.BlockSpec((1, 1), lambda i: (0, 0)),
        ],
        out_specs=pl.BlockSpec((BLK, 1), lambda i: (i, 0)),
        out_shape=jax.ShapeDtypeStruct((NPAD, 1), jnp.float32),
    )(p2, y2, dinv, W2, b2.reshape(1, 32), Wf1, bf1.reshape(1, 32),
      Wf2, bf2.reshape(1, 1))


def kernel(x, edge_index, W1, b1, W2, b2, Wf1, bf1, Wf2, bf2):
    src = edge_index[0].astype(jnp.int32)
    dst = edge_index[1].astype(jnp.int32)
    dstA = dst.reshape(NW, A_GROUPS, G)
    srcB = src.reshape(NS, B_CH, B_GPC, G)
    dstB = dst.reshape(NS, B_CH, B_GPC, G)
    srcC = (src * 2).reshape(NS, C_CH, C_GPC, G)
    dstC = dst.reshape(NS, C_CH, C_GPC, G)
    xp = jnp.pad(x, ((0, NPAD - N), (0, 0)))

    deg2 = _sc_degree(dstA)
    dinv, y1c = _tc_prep(deg2, xp)
    p1c = _sc_prop1(y1c, srcB, dstB)
    y2 = _tc_mid(p1c, y1c, dinv, W1, b1)
    y2v = jnp.pad(y2.reshape(2 * NPAD, 16), ((0, 8), (0, 0)))
    p2 = _sc_prop16(y2v, srcC, dstC)
    outp = _tc_final(p2.reshape(NPAD, 32), y2, dinv,
                     W2, b2, Wf1, bf1, Wf2, bf2)
    return outp[:N]


# speedup vs baseline: 1.1043x; 1.1043x over previous
import functools

import jax
import jax.numpy as jnp
from jax import lax
from jax.experimental import pallas as pl
from jax.experimental.pallas import tpu as pltpu
from jax.experimental.pallas import tpu_sc as plsc

N = 100000
E = 3200000
NC = 2
NS = 16
NW = NC * NS
NPAD = 102400
TSL = NPAD // NS
G = 125
WB = 100
D = 5

A_GROUPS = E // NW // G
B_CH = 16
B_GPC = E // NS // B_CH // G
C_CH = 64
C_GPC = E // NS // C_CH // G

_mesh = plsc.VectorSubcoreMesh(core_axis_name="c", subcore_axis_name="s")
_sc_params = pltpu.CompilerParams(use_tc_tiling_on_sc=False)


def _zero_fill(ref, rows):
    def body(i, _):
        ref[i] = jnp.zeros((16,), jnp.float32)
        return 0
    lax.fori_loop(0, rows, body, 0)


def _zero_fill_1d(ref, n16):
    def body(i, _):
        ref[pl.ds(i * 16, 16)] = jnp.zeros((16,), jnp.float32)
        return 0
    lax.fori_loop(0, n16, body, 0)


def _edge_loop(src_hbm, dst_hbm, sidx, gather_fn, scatter_fn,
               srcb, dstb, gsem, ssem, isem, nch, gpc):
    pltpu.async_copy(src_hbm.at[sidx, 0], srcb.at[0], isem.at[0])
    pltpu.async_copy(dst_hbm.at[sidx, 0], dstb.at[0], isem.at[1])

    def chunk(k, _):
        b = lax.rem(k, 2)
        pltpu.make_async_copy(src_hbm.at[sidx, k], srcb.at[b],
                              isem.at[0]).wait()
        pltpu.make_async_copy(dst_hbm.at[sidx, k], dstb.at[b],
                              isem.at[1]).wait()
        @pl.when(k + 1 < nch)
        def _():
            pltpu.async_copy(src_hbm.at[sidx, k + 1], srcb.at[1 - b],
                             isem.at[0])
            pltpu.async_copy(dst_hbm.at[sidx, k + 1], dstb.at[1 - b],
                             isem.at[1])
        def quint(q, _):
            gs = [gather_fn(srcb.at[b, q * D + d], d) for d in range(D)]
            ss = []
            for d in range(D):
                gs[d].wait()
                ss.append(scatter_fn(dstb.at[b, q * D + d], d))
            for d in range(D):
                ss[d].wait()
            return 0
        lax.fori_loop(0, gpc // D, quint, 0)
        return 0
    lax.fori_loop(0, nch, chunk, 0)


def _deg_body(dst_hbm, out_hbm, acc, dst_v, ones_v, zb_v, osem):
    c = lax.axis_index("c")
    s = lax.axis_index("s")
    w = c * NS + s
    _zero_fill_1d(zb_v, TSL // 16)
    pltpu.sync_copy(zb_v, acc.at[pl.ds(s * TSL, TSL)])
    def ones_body(i, _):
        ones_v[pl.ds(i * 16, 16)] = jnp.ones((16,), jnp.float32)
        return 0
    lax.fori_loop(0, 8, ones_body, 0)
    plsc.subcore_barrier()
    pltpu.sync_copy(dst_hbm.at[w], dst_v)
    def grp(t, _):
        cs = [pltpu.async_copy(ones_v.at[pl.ds(0, G)],
                               acc.at[dst_v.at[t * 8 + d]],
                               osem.at[d], add=True)
              for d in range(8)]
        for d in range(8):
            cs[d].wait()
        return 0
    lax.fori_loop(0, A_GROUPS // 8, grp, 0)
    plsc.subcore_barrier()
    pltpu.sync_copy(acc.at[pl.ds(s * TSL, TSL)], zb_v)
    pltpu.sync_copy(zb_v, out_hbm.at[c, pl.ds(s * TSL, TSL)])


@functools.partial(
    pl.kernel,
    out_type=jax.ShapeDtypeStruct((NC, NPAD), jnp.float32),
    mesh=_mesh,
    compiler_params=_sc_params,
    scratch_types=[
        pltpu.VMEM_SHARED((NPAD,), jnp.float32),
        pltpu.VMEM((A_GROUPS, G), jnp.int32),
        pltpu.VMEM((128,), jnp.float32),
        pltpu.VMEM((TSL,), jnp.float32),
        pltpu.SemaphoreType.DMA((8,)),
    ],
)
def _sc_degree(dst_hbm, out_hbm, acc, dst_v, ones_v, zb_v, osem):
    _deg_body(dst_hbm, out_hbm, acc, dst_v, ones_v, zb_v, osem)


def _p1_body(y1c_hbm, src_hbm, dst_hbm, out_hbm, tbl, acc,
             srcb, dstb, rows_v, yb_v, zb_v, gsem, ssem, isem):
    c = lax.axis_index("c")
    s = lax.axis_index("s")
    pltpu.sync_copy(y1c_hbm.at[c, pl.ds(s * TSL, TSL)], yb_v)
    pltpu.sync_copy(yb_v, tbl.at[pl.ds(s * TSL, TSL)])
    _zero_fill_1d(zb_v, TSL // 16)
    pltpu.sync_copy(zb_v, acc.at[pl.ds(s * TSL, TSL)])
    plsc.subcore_barrier()
    def gather(idx, d):
        return pltpu.async_copy(tbl.at[idx], rows_v.at[d], gsem.at[d])
    def scatter(idx, d):
        return pltpu.async_copy(rows_v.at[d], acc.at[idx], ssem.at[d],
                                add=True)
    _edge_loop(src_hbm, dst_hbm, s, gather, scatter,
               srcb, dstb, gsem, ssem, isem, B_CH, B_GPC)
    plsc.subcore_barrier()
    pltpu.sync_copy(acc.at[pl.ds(s * TSL, TSL)], zb_v)
    pltpu.sync_copy(zb_v, out_hbm.at[c, pl.ds(s * TSL, TSL)])


@functools.partial(
    pl.kernel,
    out_type=jax.ShapeDtypeStruct((NC, NPAD), jnp.float32),
    mesh=_mesh,
    compiler_params=_sc_params,
    scratch_types=[
        pltpu.VMEM_SHARED((NPAD,), jnp.float32),
        pltpu.VMEM_SHARED((NPAD,), jnp.float32),
        pltpu.VMEM((2, B_GPC, G), jnp.int32),
        pltpu.VMEM((2, B_GPC, G), jnp.int32),
        pltpu.VMEM((D, G), jnp.float32),
        pltpu.VMEM((TSL,), jnp.float32),
        pltpu.VMEM((TSL,), jnp.float32),
        pltpu.SemaphoreType.DMA((D,)),
        pltpu.SemaphoreType.DMA((D,)),
        pltpu.SemaphoreType.DMA((2,)),
    ],
)
def _sc_prop1(y1c_hbm, src_hbm, dst_hbm, out_hbm, tbl, acc,
              srcb, dstb, rows_v, yb_v, zb_v, gsem, ssem, isem):
    _p1_body(y1c_hbm, src_hbm, dst_hbm, out_hbm, tbl, acc,
             srcb, dstb, rows_v, yb_v, zb_v, gsem, ssem, isem)


def _p2_body(y2v_hbm, src_hbm, dst_hbm, out_hbm, acc,
             srcb, dstb, rows_v, zb_v, gsem, ssem, isem):
    c = lax.axis_index("c")
    s = lax.axis_index("s")
    tbl = y2v_hbm.at[pl.ds(c, 2 * N)]
    _zero_fill(zb_v, WB)
    def zinit(t, _):
        pltpu.sync_copy(zb_v, acc.at[pl.ds(s * TSL + t * WB, WB)])
        return 0
    lax.fori_loop(0, TSL // WB, zinit, 0)
    plsc.subcore_barrier()
    def gather(idx, d):
        return pltpu.async_copy(tbl.at[idx], rows_v.at[d], gsem.at[d])
    def scatter(idx, d):
        return pltpu.async_copy(rows_v.at[d], acc.at[idx], ssem.at[d],
                                add=True)
    _edge_loop(src_hbm, dst_hbm, s, gather, scatter,
               srcb, dstb, gsem, ssem, isem, C_CH, C_GPC)
    plsc.subcore_barrier()
    def wback(t, _):
        pltpu.sync_copy(acc.at[pl.ds(s * TSL + t * WB, WB)], zb_v)
        pltpu.sync_copy(zb_v, out_hbm.at[pl.ds(s * TSL + t * WB, WB), c])
        return 0
    lax.fori_loop(0, TSL // WB, wback, 0)


@functools.partial(
    pl.kernel,
    out_type=jax.ShapeDtypeStruct((NPAD, NC, 16), jnp.float32),
    mesh=_mesh,
    compiler_params=_sc_params,
    scratch_types=[
        pltpu.VMEM_SHARED((NPAD, 16), jnp.float32),
        pltpu.VMEM((2, C_GPC, G), jnp.int32),
        pltpu.VMEM((2, C_GPC, G), jnp.int32),
        pltpu.VMEM((D, G, 16), jnp.float32),
        pltpu.VMEM((WB, 16), jnp.float32),
        pltpu.SemaphoreType.DMA((D,)),
        pltpu.SemaphoreType.DMA((D,)),
        pltpu.SemaphoreType.DMA((2,)),
    ],
)
def _sc_prop16(y2v_hbm, src_hbm, dst_hbm, out_hbm, acc,
               srcb, dstb, rows_v, zb_v, gsem, ssem, isem):
    _p2_body(y2v_hbm, src_hbm, dst_hbm, out_hbm, acc,
             srcb, dstb, rows_v, zb_v, gsem, ssem, isem)


BLK = 6400
GRID = NPAD // BLK
BL8 = BLK // 8


def _tc_prep_body(deg2_ref, x_ref, dinv_ref, y1c_ref):
    d = deg2_ref[0, :] + deg2_ref[1, :] + 1.0
    dv = lax.rsqrt(d)
    dinv_ref[0, :] = dv
    y1c_ref[0, :] = x_ref[:, 0] * dv
    y1c_ref[1, :] = x_ref[:, 1] * dv


def _tc_prep(deg2, xp):
    return pl.pallas_call(
        _tc_prep_body,
        grid=(GRID,),
        in_specs=[
            pl.BlockSpec((NC, BLK), lambda i: (0, i)),
            pl.BlockSpec((BLK, 2), lambda i: (i, 0)),
        ],
        out_specs=[
            pl.BlockSpec((1, BLK), lambda i: (0, i)),
            pl.BlockSpec((NC, BLK), lambda i: (0, i)),
        ],
        out_shape=[
            jax.ShapeDtypeStruct((1, NPAD), jnp.float32),
            jax.ShapeDtypeStruct((NC, NPAD), jnp.float32),
        ],
    )(deg2, xp)


def _tc_mid_body(p1c_ref, y1c_ref, dinv_ref, W1_ref, b1_ref, y2_ref):
    dv = dinv_ref[0, :]
    prop0 = dv * (p1c_ref[0, :] + y1c_ref[0, :])
    prop1 = dv * (p1c_ref[1, :] + y1c_ref[1, :])
    h = (prop0[:, None] * W1_ref[0:1, :]
         + prop1[:, None] * W1_ref[1:2, :]
         + b1_ref[...])
    h = jnp.maximum(h, 0.0)
    y2_ref[...] = h * dv[:, None]


def _tc_mid(p1c, y1c, dinv, W1, b1):
    return pl.pallas_call(
        _tc_mid_body,
        grid=(GRID,),
        in_specs=[
            pl.BlockSpec((NC, BLK), lambda i: (0, i)),
            pl.BlockSpec((NC, BLK), lambda i: (0, i)),
            pl.BlockSpec((1, BLK), lambda i: (0, i)),
            pl.BlockSpec((2, 32), lambda i: (0, 0)),
            pl.BlockSpec((1, 32), lambda i: (0, 0)),
        ],
        out_specs=pl.BlockSpec((BLK, 32), lambda i: (i, 0)),
        out_shape=jax.ShapeDtypeStruct((NPAD, 32), jnp.float32),
    )(p1c, y1c, dinv, W1, b1.reshape(1, 32))


def _tc_final_body(p2_ref, y2_ref, dinv_ref, W2_ref, b2_ref,
                   Wf1_ref, bf1_ref, Wf2_ref, bf2_ref, out_ref):
    dv = dinv_ref[0, :][:, None]
    prop2 = dv * (p2_ref[...] + y2_ref[...])
    h2 = jnp.maximum(
        jnp.dot(prop2, W2_ref[...], preferred_element_type=jnp.float32)
        + b2_ref[...], 0.0)
    h3 = jnp.maximum(
        jnp.dot(h2, Wf1_ref[...], preferred_element_type=jnp.float32)
        + bf1_ref[...], 0.0)
    out_ref[...] = (
        jnp.dot(h3, Wf2_ref[...], preferred_element_type=jnp.float32)
        + bf2_ref[...])


def _tc_final(p2, y2, dinv, W2, b2, Wf1, bf1, Wf2, bf2):
    return pl.pallas_call(
        _tc_final_body,
        grid=(GRID,),
        in_specs=[
            pl.BlockSpec((BLK, 32), lambda i: (i, 0)),
            pl.BlockSpec((BLK, 32), lambda i: (i, 0)),
            pl.BlockSpec((1, BLK), lambda i: (0, i)),
            pl.BlockSpec((32, 32), lambda i: (0, 0)),
            pl.BlockSpec((1, 32), lambda i: (0, 0)),
            pl.BlockSpec((32, 32), lambda i: (0, 0)),
            pl.BlockSpec((1, 32), lambda i: (0, 0)),
            pl.BlockSpec((32, 1), lambda i: (0, 0)),
            pl.BlockSpec((1, 1), lambda i: (0, 0)),
        ],
        out_specs=pl.BlockSpec((BLK, 1), lambda i: (i, 0)),
        out_shape=jax.ShapeDtypeStruct((NPAD, 1), jnp.float32),
    )(p2, y2, dinv, W2, b2.reshape(1, 32), Wf1, bf1.reshape(1, 32),
      Wf2, bf2.reshape(1, 1))


def kernel(x, edge_index, W1, b1, W2, b2, Wf1, bf1, Wf2, bf2):
    src = edge_index[0].astype(jnp.int32)
    dst = edge_index[1].astype(jnp.int32)
    dstA = dst.reshape(NW, A_GROUPS, G)
    srcB = src.reshape(NS, B_CH, B_GPC, G)
    dstB = dst.reshape(NS, B_CH, B_GPC, G)
    srcC = (src * 2).reshape(NS, C_CH, C_GPC, G)
    dstC = dst.reshape(NS, C_CH, C_GPC, G)
    xp = jnp.pad(x, ((0, NPAD - N), (0, 0)))

    deg2 = _sc_degree(dstA)
    dinv, y1c = _tc_prep(deg2, xp)
    p1c = _sc_prop1(y1c, srcB, dstB)
    y2 = _tc_mid(p1c, y1c, dinv, W1, b1)
    p2 = _sc_prop16(y2.reshape(2 * NPAD, 16), srcC, dstC)
    outp = _tc_final(p2.reshape(NPAD, 32), y2, dinv,
                     W2, b2, Wf1, bf1, Wf2, bf2)
    return outp[:N]


# speedup vs baseline: 1.1645x; 1.0545x over previous
import functools

import jax
import jax.numpy as jnp
from jax import lax
from jax.experimental import pallas as pl
from jax.experimental.pallas import tpu as pltpu
from jax.experimental.pallas import tpu_sc as plsc

N = 100000
E = 3200000
NC = 2
NS = 16
NW = NC * NS
NPAD = 102400
TSL = NPAD // NS
G = 125
WB = 100
D = 5

A_GROUPS = E // NW // G
B_CH = 16
B_GPC = E // NS // B_CH // G
C_CH = 64
C_GPC = E // NS // C_CH // G

_mesh = plsc.VectorSubcoreMesh(core_axis_name="c", subcore_axis_name="s")
_sc_params = pltpu.CompilerParams(use_tc_tiling_on_sc=False)
_sc_params_nl = pltpu.CompilerParams(use_tc_tiling_on_sc=False,
                                     needs_layout_passes=False)


def _rsqrt16(d):
    i = plsc.bitcast(d, jnp.int32)
    i = jnp.int32(0x5F3759DF) - (i >> 1)
    y = plsc.bitcast(i, jnp.float32)
    for _ in range(3):
        y = y * (1.5 - 0.5 * d * y * y)
    return y


def _zero_fill(ref, rows):
    def body(i, _):
        ref[i] = jnp.zeros((16,), jnp.float32)
        return 0
    lax.fori_loop(0, rows, body, 0)


def _zero_fill_1d(ref, n16):
    def body(i, _):
        ref[pl.ds(i * 16, 16)] = jnp.zeros((16,), jnp.float32)
        return 0
    lax.fori_loop(0, n16, body, 0)


def _edge_loop(src_hbm, dst_hbm, sidx, gather_fn, scatter_fn,
               srcb, dstb, gsem, ssem, isem, nch, gpc):
    pltpu.async_copy(src_hbm.at[sidx, 0], srcb.at[0], isem.at[0])
    pltpu.async_copy(dst_hbm.at[sidx, 0], dstb.at[0], isem.at[1])

    def chunk(k, _):
        b = lax.rem(k, 2)
        pltpu.make_async_copy(src_hbm.at[sidx, k], srcb.at[b],
                              isem.at[0]).wait()
        pltpu.make_async_copy(dst_hbm.at[sidx, k], dstb.at[b],
                              isem.at[1]).wait()
        @pl.when(k + 1 < nch)
        def _():
            pltpu.async_copy(src_hbm.at[sidx, k + 1], srcb.at[1 - b],
                             isem.at[0])
            pltpu.async_copy(dst_hbm.at[sidx, k + 1], dstb.at[1 - b],
                             isem.at[1])
        def quint(q, _):
            gs = [gather_fn(srcb.at[b, q * D + d], d) for d in range(D)]
            ss = []
            for d in range(D):
                gs[d].wait()
                ss.append(scatter_fn(dstb.at[b, q * D + d], d))
            for d in range(D):
                ss[d].wait()
            return 0
        lax.fori_loop(0, gpc // D, quint, 0)
        return 0
    lax.fori_loop(0, nch, chunk, 0)


OF = 10


def _ab_body(xc_hbm, src_hbm, dst_hbm, p1_hbm, dinv_hbm,
             dega, tbl, acc, srcb, dstb, ones_v, rows_v, yb_v, zb_v, dv_v,
             osem, gsem, ssem, isem):
    c = lax.axis_index("c")
    s = lax.axis_index("s")
    sl = pl.ds(s * TSL, TSL)
    _zero_fill_1d(zb_v, TSL // 16)
    pltpu.sync_copy(zb_v, dega.at[sl])
    pltpu.sync_copy(zb_v, acc.at[sl])
    def ones_body(i, _):
        ones_v[pl.ds(i * 16, 16)] = jnp.ones((16,), jnp.float32)
        return 0
    lax.fori_loop(0, 8, ones_body, 0)
    plsc.subcore_barrier()
    pltpu.async_copy(dst_hbm.at[s, 0], dstb.at[0], isem.at[1])
    def cchunk(k, _):
        b = lax.rem(k, 2)
        pltpu.make_async_copy(dst_hbm.at[s, k], dstb.at[b], isem.at[1]).wait()
        @pl.when(k + 1 < B_CH)
        def _():
            pltpu.async_copy(dst_hbm.at[s, k + 1], dstb.at[1 - b], isem.at[1])
        def grp(t, _):
            cs = [pltpu.async_copy(ones_v.at[pl.ds(0, G)],
                                   dega.at[dstb.at[b, t * OF + d]],
                                   osem.at[d], add=True)
                  for d in range(OF)]
            for d in range(OF):
                cs[d].wait()
            return 0
        lax.fori_loop(0, B_GPC // OF, grp, 0)
        return 0
    lax.fori_loop(0, B_CH, cchunk, 0)
    plsc.subcore_barrier()
    pltpu.sync_copy(dega.at[sl], zb_v)
    pltpu.sync_copy(xc_hbm.at[c, sl], yb_v)
    def rs(i, _):
        w = pl.ds(i * 16, 16)
        y = _rsqrt16(zb_v[w] + 1.0)
        dv_v[w] = y
        yb_v[w] = yb_v[w] * y
        return 0
    lax.fori_loop(0, TSL // 16, rs, 0)
    pltpu.sync_copy(yb_v, tbl.at[sl])
    @pl.when(c == 0)
    def _():
        pltpu.sync_copy(dv_v, dinv_hbm.at[0, sl])
    plsc.subcore_barrier()
    def gather(idx, d):
        return pltpu.async_copy(tbl.at[idx], rows_v.at[d], gsem.at[d])
    def scatter(idx, d):
        return pltpu.async_copy(rows_v.at[d], acc.at[idx], ssem.at[d],
                                add=True)
    _edge_loop(src_hbm, dst_hbm, s, gather, scatter,
               srcb, dstb, gsem, ssem, isem, B_CH, B_GPC)
    plsc.subcore_barrier()
    pltpu.sync_copy(acc.at[sl], zb_v)
    pltpu.sync_copy(zb_v, p1_hbm.at[c, sl])


@functools.partial(
    pl.kernel,
    out_type=[
        jax.ShapeDtypeStruct((NC, NPAD), jnp.float32),
        jax.ShapeDtypeStruct((1, NPAD), jnp.float32),
    ],
    mesh=_mesh,
    compiler_params=_sc_params_nl,
    scratch_types=[
        pltpu.VMEM_SHARED((NPAD,), jnp.float32),
        pltpu.VMEM_SHARED((NPAD,), jnp.float32),
        pltpu.VMEM_SHARED((NPAD,), jnp.float32),
        pltpu.VMEM((2, B_GPC, G), jnp.int32),
        pltpu.VMEM((2, B_GPC, G), jnp.int32),
        pltpu.VMEM((128,), jnp.float32),
        pltpu.VMEM((D, G), jnp.float32),
        pltpu.VMEM((TSL,), jnp.float32),
        pltpu.VMEM((TSL,), jnp.float32),
        pltpu.VMEM((TSL,), jnp.float32),
        pltpu.SemaphoreType.DMA((OF,)),
        pltpu.SemaphoreType.DMA((D,)),
        pltpu.SemaphoreType.DMA((D,)),
        pltpu.SemaphoreType.DMA((2,)),
    ],
)
def _sc_ab(xc_hbm, src_hbm, dst_hbm, p1_hbm, dinv_hbm,
           dega, tbl, acc, srcb, dstb, ones_v, rows_v, yb_v, zb_v, dv_v,
           osem, gsem, ssem, isem):
    _ab_body(xc_hbm, src_hbm, dst_hbm, p1_hbm, dinv_hbm,
             dega, tbl, acc, srcb, dstb, ones_v, rows_v, yb_v, zb_v, dv_v,
             osem, gsem, ssem, isem)


def _p2_body(y2v_hbm, src_hbm, dst_hbm, out_hbm, acc,
             srcb, dstb, rows_v, zb_v, gsem, ssem, isem):
    c = lax.axis_index("c")
    s = lax.axis_index("s")
    tbl = y2v_hbm.at[pl.ds(c, 2 * N)]
    _zero_fill(zb_v, WB)
    def zinit(t, _):
        pltpu.sync_copy(zb_v, acc.at[pl.ds(s * TSL + t * WB, WB)])
        return 0
    lax.fori_loop(0, TSL // WB, zinit, 0)
    plsc.subcore_barrier()
    def gather(idx, d):
        return pltpu.async_copy(tbl.at[idx], rows_v.at[d], gsem.at[d])
    def scatter(idx, d):
        return pltpu.async_copy(rows_v.at[d], acc.at[idx], ssem.at[d],
                                add=True)
    _edge_loop(src_hbm, dst_hbm, s, gather, scatter,
               srcb, dstb, gsem, ssem, isem, C_CH, C_GPC)
    plsc.subcore_barrier()
    def wback(t, _):
        pltpu.sync_copy(acc.at[pl.ds(s * TSL + t * WB, WB)], zb_v)
        pltpu.sync_copy(zb_v, out_hbm.at[pl.ds(s * TSL + t * WB, WB), c])
        return 0
    lax.fori_loop(0, TSL // WB, wback, 0)


@functools.partial(
    pl.kernel,
    out_type=jax.ShapeDtypeStruct((NPAD, NC, 16), jnp.float32),
    mesh=_mesh,
    compiler_params=_sc_params,
    scratch_types=[
        pltpu.VMEM_SHARED((NPAD, 16), jnp.float32),
        pltpu.VMEM((2, C_GPC, G), jnp.int32),
        pltpu.VMEM((2, C_GPC, G), jnp.int32),
        pltpu.VMEM((D, G, 16), jnp.float32),
        pltpu.VMEM((WB, 16), jnp.float32),
        pltpu.SemaphoreType.DMA((D,)),
        pltpu.SemaphoreType.DMA((D,)),
        pltpu.SemaphoreType.DMA((2,)),
    ],
)
def _sc_prop16(y2v_hbm, src_hbm, dst_hbm, out_hbm, acc,
               srcb, dstb, rows_v, zb_v, gsem, ssem, isem):
    _p2_body(y2v_hbm, src_hbm, dst_hbm, out_hbm, acc,
             srcb, dstb, rows_v, zb_v, gsem, ssem, isem)


BLK = 6400
GRID = NPAD // BLK
BL8 = BLK // 8


def _tc_mid_body(p1c_ref, dinv_ref, xc_ref, W1_ref, b1_ref, y2_ref):
    dv = dinv_ref[0, :]
    prop0 = dv * (p1c_ref[0, :] + dv * xc_ref[0, :])
    prop1 = dv * (p1c_ref[1, :] + dv * xc_ref[1, :])
    h = (prop0[:, None] * W1_ref[0:1, :]
         + prop1[:, None] * W1_ref[1:2, :]
         + b1_ref[...])
    h = jnp.maximum(h, 0.0)
    y2_ref[...] = h * dv[:, None]


def _tc_mid(p1c, dinv, xc, W1, b1):
    return pl.pallas_call(
        _tc_mid_body,
        grid=(GRID,),
        in_specs=[
            pl.BlockSpec((NC, BLK), lambda i: (0, i)),
            pl.BlockSpec((1, BLK), lambda i: (0, i)),
            pl.BlockSpec((NC, BLK), lambda i: (0, i)),
            pl.BlockSpec((2, 32), lambda i: (0, 0)),
            pl.BlockSpec((1, 32), lambda i: (0, 0)),
        ],
        out_specs=pl.BlockSpec((BLK, 32), lambda i: (i, 0)),
        out_shape=jax.ShapeDtypeStruct((NPAD, 32), jnp.float32),
    )(p1c, dinv, xc, W1, b1.reshape(1, 32))


def _tc_final_body(p2_ref, y2_ref, dinv_ref, W2_ref, b2_ref,
                   Wf1_ref, bf1_ref, Wf2_ref, bf2_ref, out_ref):
    dv = dinv_ref[0, :][:, None]
    prop2 = dv * (p2_ref[...] + y2_ref[...])
    h2 = jnp.maximum(
        jnp.dot(prop2, W2_ref[...], preferred_element_type=jnp.float32)
        + b2_ref[...], 0.0)
    h3 = jnp.maximum(
        jnp.dot(h2, Wf1_ref[...], preferred_element_type=jnp.float32)
        + bf1_ref[...], 0.0)
    out_ref[...] = (
        jnp.dot(h3, Wf2_ref[...], preferred_element_type=jnp.float32)
        + bf2_ref[...])


def _tc_final(p2, y2, dinv, W2, b2, Wf1, bf1, Wf2, bf2):
    return pl.pallas_call(
        _tc_final_body,
        grid=(GRID,),
        in_specs=[
            pl.BlockSpec((BLK, 32), lambda i: (i, 0)),
            pl.BlockSpec((BLK, 32), lambda i: (i, 0)),
            pl.BlockSpec((1, BLK), lambda i: (0, i)),
            pl.BlockSpec((32, 32), lambda i: (0, 0)),
            pl.BlockSpec((1, 32), lambda i: (0, 0)),
            pl.BlockSpec((32, 32), lambda i: (0, 0)),
            pl.BlockSpec((1, 32), lambda i: (0, 0)),
            pl.BlockSpec((32, 1), lambda i: (0, 0)),
            pl.BlockSpec((1, 1), lambda i: (0, 0)),
        ],
        out_specs=pl.BlockSpec((BLK, 1), lambda i: (i, 0)),
        out_shape=jax.ShapeDtypeStruct((NPAD, 1), jnp.float32),
    )(p2, y2, dinv, W2, b2.reshape(1, 32), Wf1, bf1.reshape(1, 32),
      Wf2, bf2.reshape(1, 1))


def kernel(x, edge_index, W1, b1, W2, b2, Wf1, bf1, Wf2, bf2):
    src = edge_index[0].astype(jnp.int32)
    dst = edge_index[1].astype(jnp.int32)
    srcB = src.reshape(NS, B_CH, B_GPC, G)
    dstB = dst.reshape(NS, B_CH, B_GPC, G)
    srcC = (src * 2).reshape(NS, C_CH, C_GPC, G)
    dstC = dst.reshape(NS, C_CH, C_GPC, G)
    xc = jnp.pad(x.T, ((0, 0), (0, NPAD - N)))

    p1c, dinv = _sc_ab(xc, srcB, dstB)
    y2 = _tc_mid(p1c, dinv, xc, W1, b1)
    p2 = _sc_prop16(y2.reshape(2 * NPAD, 16), srcC, dstC)
    outp = _tc_final(p2.reshape(NPAD, 32), y2, dinv,
                     W2, b2, Wf1, bf1, Wf2, bf2)
    return outp[:N]


# speedup vs baseline: 1.1683x; 1.0033x over previous
import functools

import jax
import jax.numpy as jnp
from jax import lax
from jax.experimental import pallas as pl
from jax.experimental.pallas import tpu as pltpu
from jax.experimental.pallas import tpu_sc as plsc

N = 100000
E = 3200000
NC = 2
NS = 16
NW = NC * NS
NPAD = 102400
TSL = NPAD // NS
G = 125
WB = 100
D = 5

A_GROUPS = E // NW // G
B_CH = 16
B_GPC = E // NS // B_CH // G
C_CH = 64
C_GPC = E // NS // C_CH // G

_mesh = plsc.VectorSubcoreMesh(core_axis_name="c", subcore_axis_name="s")
_sc_params = pltpu.CompilerParams(use_tc_tiling_on_sc=False)
_sc_params_nl = pltpu.CompilerParams(use_tc_tiling_on_sc=False,
                                     needs_layout_passes=False)


def _rsqrt16(d):
    i = plsc.bitcast(d, jnp.int32)
    i = jnp.int32(0x5F3759DF) - (i >> 1)
    y = plsc.bitcast(i, jnp.float32)
    for _ in range(3):
        y = y * (1.5 - 0.5 * d * y * y)
    return y


def _zero_fill(ref, rows):
    def body(i, _):
        ref[i] = jnp.zeros((16,), jnp.float32)
        return 0
    lax.fori_loop(0, rows, body, 0)


def _zero_fill_1d(ref, n16):
    def body(i, _):
        ref[pl.ds(i * 16, 16)] = jnp.zeros((16,), jnp.float32)
        return 0
    lax.fori_loop(0, n16, body, 0)


def _edge_loop(src_hbm, dst_hbm, sidx, gather_fn, scatter_fn,
               srcb, dstb, gsem, ssem, isem, nch, gpc):
    pltpu.async_copy(src_hbm.at[sidx, 0], srcb.at[0], isem.at[0])
    pltpu.async_copy(dst_hbm.at[sidx, 0], dstb.at[0], isem.at[1])

    def chunk(k, _):
        b = lax.rem(k, 2)
        pltpu.make_async_copy(src_hbm.at[sidx, k], srcb.at[b],
                              isem.at[0]).wait()
        pltpu.make_async_copy(dst_hbm.at[sidx, k], dstb.at[b],
                              isem.at[1]).wait()
        @pl.when(k + 1 < nch)
        def _():
            pltpu.async_copy(src_hbm.at[sidx, k + 1], srcb.at[1 - b],
                             isem.at[0])
            pltpu.async_copy(dst_hbm.at[sidx, k + 1], dstb.at[1 - b],
                             isem.at[1])
        def quint(q, _):
            gs = [gather_fn(srcb.at[b, q * D + d], d) for d in range(D)]
            ss = []
            for d in range(D):
                gs[d].wait()
                ss.append(scatter_fn(dstb.at[b, q * D + d], d))
            for d in range(D):
                ss[d].wait()
            return 0
        lax.fori_loop(0, gpc // D, quint, 0)
        return 0
    lax.fori_loop(0, nch, chunk, 0)


OF = 10


def _ab_body(xc_hbm, src_hbm, dst_hbm, p1_hbm, dinv_hbm,
             dega, tbl, acc, srcb, dstb, ones_v, rows_v, yb_v, zb_v, dv_v,
             osem, gsem, ssem, isem):
    c = lax.axis_index("c")
    s = lax.axis_index("s")
    sl = pl.ds(s * TSL, TSL)
    _zero_fill_1d(zb_v, TSL // 16)
    pltpu.sync_copy(zb_v, dega.at[sl])
    pltpu.sync_copy(zb_v, acc.at[sl])
    def ones_body(i, _):
        ones_v[pl.ds(i * 16, 16)] = jnp.ones((16,), jnp.float32)
        return 0
    lax.fori_loop(0, 8, ones_body, 0)
    plsc.subcore_barrier()
    pltpu.async_copy(dst_hbm.at[s, 0], dstb.at[0], isem.at[1])
    def cchunk(k, _):
        b = lax.rem(k, 2)
        pltpu.make_async_copy(dst_hbm.at[s, k], dstb.at[b], isem.at[1]).wait()
        @pl.when(k + 1 < B_CH)
        def _():
            pltpu.async_copy(dst_hbm.at[s, k + 1], dstb.at[1 - b], isem.at[1])
        def grp(t, _):
            cs = [pltpu.async_copy(ones_v.at[pl.ds(0, G)],
                                   dega.at[dstb.at[b, t * OF + d]],
                                   osem.at[d], add=True)
                  for d in range(OF)]
            for d in range(OF):
                cs[d].wait()
            return 0
        lax.fori_loop(0, B_GPC // OF, grp, 0)
        return 0
    lax.fori_loop(0, B_CH, cchunk, 0)
    plsc.subcore_barrier()
    pltpu.sync_copy(dega.at[sl], zb_v)
    pltpu.sync_copy(xc_hbm.at[c, sl], yb_v)
    def rs(i, _):
        w = pl.ds(i * 16, 16)
        y = _rsqrt16(zb_v[w] + 1.0)
        dv_v[w] = y
        yb_v[w] = yb_v[w] * y
        return 0
    lax.fori_loop(0, TSL // 16, rs, 0)
    pltpu.sync_copy(yb_v, tbl.at[sl])
    @pl.when(c == 0)
    def _():
        pltpu.sync_copy(dv_v, dinv_hbm.at[0, sl])
    plsc.subcore_barrier()
    def gather(idx, d):
        return pltpu.async_copy(tbl.at[idx], rows_v.at[d], gsem.at[d])
    def scatter(idx, d):
        return pltpu.async_copy(rows_v.at[d], acc.at[idx], ssem.at[d],
                                add=True)
    _edge_loop(src_hbm, dst_hbm, s, gather, scatter,
               srcb, dstb, gsem, ssem, isem, B_CH, B_GPC)
    plsc.subcore_barrier()
    pltpu.sync_copy(acc.at[sl], zb_v)
    pltpu.sync_copy(zb_v, p1_hbm.at[c, sl])


@functools.partial(
    pl.kernel,
    out_type=[
        jax.ShapeDtypeStruct((NC, NPAD), jnp.float32),
        jax.ShapeDtypeStruct((1, NPAD), jnp.float32),
    ],
    mesh=_mesh,
    compiler_params=_sc_params_nl,
    scratch_types=[
        pltpu.VMEM_SHARED((NPAD,), jnp.float32),
        pltpu.VMEM_SHARED((NPAD,), jnp.float32),
        pltpu.VMEM_SHARED((NPAD,), jnp.float32),
        pltpu.VMEM((2, B_GPC, G), jnp.int32),
        pltpu.VMEM((2, B_GPC, G), jnp.int32),
        pltpu.VMEM((128,), jnp.float32),
        pltpu.VMEM((D, G), jnp.float32),
        pltpu.VMEM((TSL,), jnp.float32),
        pltpu.VMEM((TSL,), jnp.float32),
        pltpu.VMEM((TSL,), jnp.float32),
        pltpu.SemaphoreType.DMA((OF,)),
        pltpu.SemaphoreType.DMA((D,)),
        pltpu.SemaphoreType.DMA((D,)),
        pltpu.SemaphoreType.DMA((2,)),
    ],
)
def _sc_ab(xc_hbm, src_hbm, dst_hbm, p1_hbm, dinv_hbm,
           dega, tbl, acc, srcb, dstb, ones_v, rows_v, yb_v, zb_v, dv_v,
           osem, gsem, ssem, isem):
    _ab_body(xc_hbm, src_hbm, dst_hbm, p1_hbm, dinv_hbm,
             dega, tbl, acc, srcb, dstb, ones_v, rows_v, yb_v, zb_v, dv_v,
             osem, gsem, ssem, isem)


def _p2_body(y2v_hbm, src_hbm, dst_hbm, out_hbm, acc,
             srcb, dstb, rows_v, zb_v, gsem, ssem, isem):
    c = lax.axis_index("c")
    s = lax.axis_index("s")
    tbl = y2v_hbm.at[pl.ds(c, 2 * N)]
    _zero_fill(zb_v, WB)
    def zinit(t, _):
        pltpu.sync_copy(zb_v, acc.at[pl.ds(s * TSL + t * WB, WB)])
        return 0
    lax.fori_loop(0, TSL // WB, zinit, 0)
    plsc.subcore_barrier()
    def gather(idx, d):
        return pltpu.async_copy(tbl.at[idx], rows_v.at[d], gsem.at[d])
    def scatter(idx, d):
        return pltpu.async_copy(rows_v.at[d], acc.at[idx], ssem.at[d],
                                add=True)
    _edge_loop(src_hbm, dst_hbm, s, gather, scatter,
               srcb, dstb, gsem, ssem, isem, C_CH, C_GPC)
    plsc.subcore_barrier()
    def wback(t, _):
        pltpu.sync_copy(acc.at[pl.ds(s * TSL + t * WB, WB)], zb_v)
        pltpu.sync_copy(zb_v, out_hbm.at[pl.ds(s * TSL + t * WB, WB), c])
        return 0
    lax.fori_loop(0, TSL // WB, wback, 0)


@functools.partial(
    pl.kernel,
    out_type=jax.ShapeDtypeStruct((NPAD, NC, 16), jnp.float32),
    mesh=_mesh,
    compiler_params=_sc_params,
    scratch_types=[
        pltpu.VMEM_SHARED((NPAD, 16), jnp.float32),
        pltpu.VMEM((2, C_GPC, G), jnp.int32),
        pltpu.VMEM((2, C_GPC, G), jnp.int32),
        pltpu.VMEM((D, G, 16), jnp.float32),
        pltpu.VMEM((WB, 16), jnp.float32),
        pltpu.SemaphoreType.DMA((D,)),
        pltpu.SemaphoreType.DMA((D,)),
        pltpu.SemaphoreType.DMA((2,)),
    ],
)
def _sc_prop16(y2v_hbm, src_hbm, dst_hbm, out_hbm, acc,
               srcb, dstb, rows_v, zb_v, gsem, ssem, isem):
    _p2_body(y2v_hbm, src_hbm, dst_hbm, out_hbm, acc,
             srcb, dstb, rows_v, zb_v, gsem, ssem, isem)


BLK = 6400
GRID = NPAD // BLK
BL8 = BLK // 8


def _tc_mid_body(p1c_ref, dinv_ref, xc_ref, W1_ref, b1_ref, y2_ref):
    dv = dinv_ref[0, :]
    prop0 = dv * (p1c_ref[0, :] + dv * xc_ref[0, :])
    prop1 = dv * (p1c_ref[1, :] + dv * xc_ref[1, :])
    h = (prop0[:, None] * W1_ref[0:1, :]
         + prop1[:, None] * W1_ref[1:2, :]
         + b1_ref[...])
    h = jnp.maximum(h, 0.0)
    y2_ref[...] = h * dv[:, None]


def _tc_mid(p1c, dinv, xc, W1, b1):
    return pl.pallas_call(
        _tc_mid_body,
        grid=(GRID,),
        in_specs=[
            pl.BlockSpec((NC, BLK), lambda i: (0, i)),
            pl.BlockSpec((1, BLK), lambda i: (0, i)),
            pl.BlockSpec((NC, BLK), lambda i: (0, i)),
            pl.BlockSpec((2, 32), lambda i: (0, 0)),
            pl.BlockSpec((1, 32), lambda i: (0, 0)),
        ],
        out_specs=pl.BlockSpec((BLK, 32), lambda i: (i, 0)),
        out_shape=jax.ShapeDtypeStruct((NPAD, 32), jnp.float32),
    )(p1c, dinv, xc, W1, b1.reshape(1, 32))


def _tc_final_body(p2_ref, y2_ref, dinv_ref, W2_ref, b2_ref,
                   Wf1_ref, bf1_ref, Wf2_ref, bf2_ref, out_ref):
    dv = dinv_ref[0, :][:, None]
    prop2 = dv * (p2_ref[...] + y2_ref[...])
    h2 = jnp.maximum(
        jnp.dot(prop2, W2_ref[...], preferred_element_type=jnp.float32)
        + b2_ref[...], 0.0)
    h3 = jnp.maximum(
        jnp.dot(h2, Wf1_ref[...], preferred_element_type=jnp.float32)
        + bf1_ref[...], 0.0)
    out_ref[...] = (
        jnp.dot(h3, Wf2_ref[...], preferred_element_type=jnp.float32)
        + bf2_ref[...])


def _tc_final(p2, y2, dinv, W2, b2, Wf1, bf1, Wf2, bf2):
    return pl.pallas_call(
        _tc_final_body,
        grid=(GRID,),
        in_specs=[
            pl.BlockSpec((BLK, 32), lambda i: (i, 0)),
            pl.BlockSpec((BLK, 32), lambda i: (i, 0)),
            pl.BlockSpec((1, BLK), lambda i: (0, i)),
            pl.BlockSpec((32, 32), lambda i: (0, 0)),
            pl.BlockSpec((1, 32), lambda i: (0, 0)),
            pl.BlockSpec((32, 32), lambda i: (0, 0)),
            pl.BlockSpec((1, 32), lambda i: (0, 0)),
            pl.BlockSpec((32, 1), lambda i: (0, 0)),
            pl.BlockSpec((1, 1), lambda i: (0, 0)),
        ],
        out_specs=pl.BlockSpec((BLK, 1), lambda i: (i, 0)),
        out_shape=jax.ShapeDtypeStruct((NPAD, 1), jnp.float32),
    )(p2, y2, dinv, W2, b2.reshape(1, 32), Wf1, bf1.reshape(1, 32),
      Wf2, bf2.reshape(1, 1))


def _tc_double_body(i_ref, o_ref):
    o_ref[...] = i_ref[...] * 2


def _tc_double(src):
    return pl.pallas_call(
        _tc_double_body,
        grid=(5,),
        in_specs=[pl.BlockSpec((E // 5 // 128, 128), lambda i: (i, 0))],
        out_specs=pl.BlockSpec((E // 5 // 128, 128), lambda i: (i, 0)),
        out_shape=jax.ShapeDtypeStruct((E // 128, 128), jnp.int32),
    )(src.reshape(E // 128, 128))


def kernel(x, edge_index, W1, b1, W2, b2, Wf1, bf1, Wf2, bf2):
    src = edge_index[0].astype(jnp.int32)
    dst = edge_index[1].astype(jnp.int32)
    srcB = src.reshape(NS, B_CH, B_GPC, G)
    dstB = dst.reshape(NS, B_CH, B_GPC, G)
    srcC = _tc_double(src).reshape(NS, C_CH, C_GPC, G)
    dstC = dst.reshape(NS, C_CH, C_GPC, G)
    xc = jnp.pad(x.T, ((0, 0), (0, NPAD - N)))

    p1c, dinv = _sc_ab(xc, srcB, dstB)
    y2 = _tc_mid(p1c, dinv, xc, W1, b1)
    p2 = _sc_prop16(y2.reshape(2 * NPAD, 16), srcC, dstC)
    outp = _tc_final(p2.reshape(NPAD, 32), y2, dinv,
                     W2, b2, Wf1, bf1, Wf2, bf2)
    return outp[:N]


# speedup vs baseline: 1.2177x; 1.0423x over previous
import functools

import jax
import jax.numpy as jnp
from jax import lax
from jax.experimental import pallas as pl
from jax.experimental.pallas import tpu as pltpu
from jax.experimental.pallas import tpu_sc as plsc

N = 100000
E = 3200000
NC = 2
NS = 16
NW = NC * NS
NPAD = 102400
TSL = NPAD // NS
G = 125
WB = 100
D = 5

A_GROUPS = E // NW // G
B_CH = 16
B_GPC = E // NS // B_CH // G
C_CH = 100
C_GPC = E // NS // C_CH // G
DC = 8

_mesh = plsc.VectorSubcoreMesh(core_axis_name="c", subcore_axis_name="s")
_sc_params = pltpu.CompilerParams(use_tc_tiling_on_sc=False)
_sc_params_nl = pltpu.CompilerParams(use_tc_tiling_on_sc=False,
                                     needs_layout_passes=False)


def _rsqrt16(d):
    i = plsc.bitcast(d, jnp.int32)
    i = jnp.int32(0x5F3759DF) - (i >> 1)
    y = plsc.bitcast(i, jnp.float32)
    for _ in range(3):
        y = y * (1.5 - 0.5 * d * y * y)
    return y


def _zero_fill(ref, rows):
    def body(i, _):
        ref[i] = jnp.zeros((16,), jnp.float32)
        return 0
    lax.fori_loop(0, rows, body, 0)


def _zero_fill_1d(ref, n16):
    def body(i, _):
        ref[pl.ds(i * 16, 16)] = jnp.zeros((16,), jnp.float32)
        return 0
    lax.fori_loop(0, n16, body, 0)


def _edge_loop(src_hbm, dst_hbm, sidx, gather_fn, scatter_fn,
               srcb, dstb, gsem, ssem, isem, nch, gpc, depth=D):
    pltpu.async_copy(src_hbm.at[sidx, 0], srcb.at[0], isem.at[0])
    pltpu.async_copy(dst_hbm.at[sidx, 0], dstb.at[0], isem.at[1])

    def chunk(k, _):
        b = lax.rem(k, 2)
        pltpu.make_async_copy(src_hbm.at[sidx, k], srcb.at[b],
                              isem.at[0]).wait()
        pltpu.make_async_copy(dst_hbm.at[sidx, k], dstb.at[b],
                              isem.at[1]).wait()
        @pl.when(k + 1 < nch)
        def _():
            pltpu.async_copy(src_hbm.at[sidx, k + 1], srcb.at[1 - b],
                             isem.at[0])
            pltpu.async_copy(dst_hbm.at[sidx, k + 1], dstb.at[1 - b],
                             isem.at[1])
        def quint(q, _):
            gs = [gather_fn(srcb.at[b, q * depth + d], d)
                  for d in range(depth)]
            ss = []
            for d in range(depth):
                gs[d].wait()
                ss.append(scatter_fn(dstb.at[b, q * depth + d], d))
            for d in range(depth):
                ss[d].wait()
            return 0
        lax.fori_loop(0, gpc // depth, quint, 0)
        return 0
    lax.fori_loop(0, nch, chunk, 0)


OF = 10


def _ab_body(xc_hbm, src_hbm, dst_hbm, p1_hbm, dinv_hbm,
             dega, tbl, acc, srcb, dstb, ones_v, rows_v, yb_v, zb_v, dv_v,
             osem, gsem, ssem, isem):
    c = lax.axis_index("c")
    s = lax.axis_index("s")
    sl = pl.ds(s * TSL, TSL)
    _zero_fill_1d(zb_v, TSL // 16)
    pltpu.sync_copy(zb_v, dega.at[sl])
    pltpu.sync_copy(zb_v, acc.at[sl])
    def ones_body(i, _):
        ones_v[pl.ds(i * 16, 16)] = jnp.ones((16,), jnp.float32)
        return 0
    lax.fori_loop(0, 8, ones_body, 0)
    plsc.subcore_barrier()
    pltpu.async_copy(dst_hbm.at[s, 0], dstb.at[0], isem.at[1])
    def cchunk(k, _):
        b = lax.rem(k, 2)
        pltpu.make_async_copy(dst_hbm.at[s, k], dstb.at[b], isem.at[1]).wait()
        @pl.when(k + 1 < B_CH)
        def _():
            pltpu.async_copy(dst_hbm.at[s, k + 1], dstb.at[1 - b], isem.at[1])
        def grp(t, _):
            cs = [pltpu.async_copy(ones_v.at[pl.ds(0, G)],
                                   dega.at[dstb.at[b, t * OF + d]],
                                   osem.at[d], add=True)
                  for d in range(OF)]
            for d in range(OF):
                cs[d].wait()
            return 0
        lax.fori_loop(0, B_GPC // OF, grp, 0)
        return 0
    lax.fori_loop(0, B_CH, cchunk, 0)
    plsc.subcore_barrier()
    pltpu.sync_copy(dega.at[sl], zb_v)
    pltpu.sync_copy(xc_hbm.at[c, sl], yb_v)
    def rs(i, _):
        w = pl.ds(i * 16, 16)
        y = _rsqrt16(zb_v[w] + 1.0)
        dv_v[w] = y
        yb_v[w] = yb_v[w] * y
        return 0
    lax.fori_loop(0, TSL // 16, rs, 0)
    pltpu.sync_copy(yb_v, tbl.at[sl])
    @pl.when(c == 0)
    def _():
        pltpu.sync_copy(dv_v, dinv_hbm.at[0, sl])
    plsc.subcore_barrier()
    def gather(idx, d):
        return pltpu.async_copy(tbl.at[idx], rows_v.at[d], gsem.at[d])
    def scatter(idx, d):
        return pltpu.async_copy(rows_v.at[d], acc.at[idx], ssem.at[d],
                                add=True)
    _edge_loop(src_hbm, dst_hbm, s, gather, scatter,
               srcb, dstb, gsem, ssem, isem, B_CH, B_GPC)
    plsc.subcore_barrier()
    pltpu.sync_copy(acc.at[sl], zb_v)
    pltpu.sync_copy(zb_v, p1_hbm.at[c, sl])


@functools.partial(
    pl.kernel,
    out_type=[
        jax.ShapeDtypeStruct((NC, NPAD), jnp.float32),
        jax.ShapeDtypeStruct((1, NPAD), jnp.float32),
    ],
    mesh=_mesh,
    compiler_params=_sc_params_nl,
    scratch_types=[
        pltpu.VMEM_SHARED((NPAD,), jnp.float32),
        pltpu.VMEM_SHARED((NPAD,), jnp.float32),
        pltpu.VMEM_SHARED((NPAD,), jnp.float32),
        pltpu.VMEM((2, B_GPC, G), jnp.int32),
        pltpu.VMEM((2, B_GPC, G), jnp.int32),
        pltpu.VMEM((128,), jnp.float32),
        pltpu.VMEM((D, G), jnp.float32),
        pltpu.VMEM((TSL,), jnp.float32),
        pltpu.VMEM((TSL,), jnp.float32),
        pltpu.VMEM((TSL,), jnp.float32),
        pltpu.SemaphoreType.DMA((OF,)),
        pltpu.SemaphoreType.DMA((D,)),
        pltpu.SemaphoreType.DMA((D,)),
        pltpu.SemaphoreType.DMA((2,)),
    ],
)
def _sc_ab(xc_hbm, src_hbm, dst_hbm, p1_hbm, dinv_hbm,
           dega, tbl, acc, srcb, dstb, ones_v, rows_v, yb_v, zb_v, dv_v,
           osem, gsem, ssem, isem):
    _ab_body(xc_hbm, src_hbm, dst_hbm, p1_hbm, dinv_hbm,
             dega, tbl, acc, srcb, dstb, ones_v, rows_v, yb_v, zb_v, dv_v,
             osem, gsem, ssem, isem)


def _p2_body(y2v_hbm, src_hbm, dst_hbm, out_hbm, acc,
             srcb, dstb, rows_v, zb_v, gsem, ssem, isem):
    c = lax.axis_index("c")
    s = lax.axis_index("s")
    tbl = y2v_hbm.at[pl.ds(c, 2 * N)]
    _zero_fill(zb_v, WB)
    def zinit(t, _):
        pltpu.sync_copy(zb_v, acc.at[pl.ds(s * TSL + t * WB, WB)])
        return 0
    lax.fori_loop(0, TSL // WB, zinit, 0)
    plsc.subcore_barrier()
    def gather(idx, d):
        return pltpu.async_copy(tbl.at[idx], rows_v.at[d], gsem.at[d])
    def scatter(idx, d):
        return pltpu.async_copy(rows_v.at[d], acc.at[idx], ssem.at[d],
                                add=True)
    _edge_loop(src_hbm, dst_hbm, s, gather, scatter,
               srcb, dstb, gsem, ssem, isem, C_CH, C_GPC, depth=DC)
    plsc.subcore_barrier()
    def wback(t, _):
        pltpu.sync_copy(acc.at[pl.ds(s * TSL + t * WB, WB)], zb_v)
        pltpu.sync_copy(zb_v, out_hbm.at[pl.ds(s * TSL + t * WB, WB), c])
        return 0
    lax.fori_loop(0, TSL // WB, wback, 0)


@functools.partial(
    pl.kernel,
    out_type=jax.ShapeDtypeStruct((NPAD, NC, 16), jnp.float32),
    mesh=_mesh,
    compiler_params=_sc_params,
    scratch_types=[
        pltpu.VMEM_SHARED((NPAD, 16), jnp.float32),
        pltpu.VMEM((2, C_GPC, G), jnp.int32),
        pltpu.VMEM((2, C_GPC, G), jnp.int32),
        pltpu.VMEM((DC, G, 16), jnp.float32),
        pltpu.VMEM((WB, 16), jnp.float32),
        pltpu.SemaphoreType.DMA((DC,)),
        pltpu.SemaphoreType.DMA((DC,)),
        pltpu.SemaphoreType.DMA((2,)),
    ],
)
def _sc_prop16(y2v_hbm, src_hbm, dst_hbm, out_hbm, acc,
               srcb, dstb, rows_v, zb_v, gsem, ssem, isem):
    _p2_body(y2v_hbm, src_hbm, dst_hbm, out_hbm, acc,
             srcb, dstb, rows_v, zb_v, gsem, ssem, isem)


BLK = 6400
GRID = NPAD // BLK
BL8 = BLK // 8


def _tc_mid_body(p1c_ref, dinv_ref, xc_ref, W1_ref, b1_ref, y2_ref):
    dv = dinv_ref[0, :]
    prop0 = dv * (p1c_ref[0, :] + dv * xc_ref[0, :])
    prop1 = dv * (p1c_ref[1, :] + dv * xc_ref[1, :])
    h = (prop0[:, None] * W1_ref[0:1, :]
         + prop1[:, None] * W1_ref[1:2, :]
         + b1_ref[...])
    h = jnp.maximum(h, 0.0)
    y2_ref[...] = h * dv[:, None]


def _tc_mid(p1c, dinv, xc, W1, b1):
    return pl.pallas_call(
        _tc_mid_body,
        grid=(GRID,),
        in_specs=[
            pl.BlockSpec((NC, BLK), lambda i: (0, i)),
            pl.BlockSpec((1, BLK), lambda i: (0, i)),
            pl.BlockSpec((NC, BLK), lambda i: (0, i)),
            pl.BlockSpec((2, 32), lambda i: (0, 0)),
            pl.BlockSpec((1, 32), lambda i: (0, 0)),
        ],
        out_specs=pl.BlockSpec((BLK, 32), lambda i: (i, 0)),
        out_shape=jax.ShapeDtypeStruct((NPAD, 32), jnp.float32),
    )(p1c, dinv, xc, W1, b1.reshape(1, 32))


def _tc_final_body(p2_ref, y2_ref, dinv_ref, W2_ref, b2_ref,
                   Wf1_ref, bf1_ref, Wf2_ref, bf2_ref, out_ref):
    dv = dinv_ref[0, :][:, None]
    prop2 = dv * (p2_ref[...] + y2_ref[...])
    h2 = jnp.maximum(
        jnp.dot(prop2, W2_ref[...], preferred_element_type=jnp.float32)
        + b2_ref[...], 0.0)
    h3 = jnp.maximum(
        jnp.dot(h2, Wf1_ref[...], preferred_element_type=jnp.float32)
        + bf1_ref[...], 0.0)
    out_ref[...] = (
        jnp.dot(h3, Wf2_ref[...], preferred_element_type=jnp.float32)
        + bf2_ref[...])


def _tc_final(p2, y2, dinv, W2, b2, Wf1, bf1, Wf2, bf2):
    return pl.pallas_call(
        _tc_final_body,
        grid=(GRID,),
        in_specs=[
            pl.BlockSpec((BLK, 32), lambda i: (i, 0)),
            pl.BlockSpec((BLK, 32), lambda i: (i, 0)),
            pl.BlockSpec((1, BLK), lambda i: (0, i)),
            pl.BlockSpec((32, 32), lambda i: (0, 0)),
            pl.BlockSpec((1, 32), lambda i: (0, 0)),
            pl.BlockSpec((32, 32), lambda i: (0, 0)),
            pl.BlockSpec((1, 32), lambda i: (0, 0)),
            pl.BlockSpec((32, 1), lambda i: (0, 0)),
            pl.BlockSpec((1, 1), lambda i: (0, 0)),
        ],
        out_specs=pl.BlockSpec((BLK, 1), lambda i: (i, 0)),
        out_shape=jax.ShapeDtypeStruct((NPAD, 1), jnp.float32),
    )(p2, y2, dinv, W2, b2.reshape(1, 32), Wf1, bf1.reshape(1, 32),
      Wf2, bf2.reshape(1, 1))


def _tc_double_body(i_ref, o_ref):
    o_ref[...] = i_ref[...] * 2


def _tc_double(src):
    return pl.pallas_call(
        _tc_double_body,
        grid=(5,),
        in_specs=[pl.BlockSpec((E // 5 // 128, 128), lambda i: (i, 0))],
        out_specs=pl.BlockSpec((E // 5 // 128, 128), lambda i: (i, 0)),
        out_shape=jax.ShapeDtypeStruct((E // 128, 128), jnp.int32),
    )(src.reshape(E // 128, 128))


def kernel(x, edge_index, W1, b1, W2, b2, Wf1, bf1, Wf2, bf2):
    src = edge_index[0].astype(jnp.int32)
    dst = edge_index[1].astype(jnp.int32)
    srcB = src.reshape(NS, B_CH, B_GPC, G)
    dstB = dst.reshape(NS, B_CH, B_GPC, G)
    srcC = _tc_double(src).reshape(NS, C_CH, C_GPC, G)
    dstC = dst.reshape(NS, C_CH, C_GPC, G)
    xc = jnp.pad(x.T, ((0, 0), (0, NPAD - N)))

    p1c, dinv = _sc_ab(xc, srcB, dstB)
    y2 = _tc_mid(p1c, dinv, xc, W1, b1)
    p2 = _sc_prop16(y2.reshape(2 * NPAD, 16), srcC, dstC)
    outp = _tc_final(p2.reshape(NPAD, 32), y2, dinv,
                     W2, b2, Wf1, bf1, Wf2, bf2)
    return outp[:N]
